# baseline (speedup 1.0000x reference)
import jax
import jax.numpy as jnp
from jax import lax
from jax.experimental import pallas as pl
from jax.experimental.pallas import tpu as pltpu
from jax.experimental.pallas import tpu_sc as plsc

L = 16
NC = 2
NS = 16
NW = NC * NS
N_COMP = 5
C_DIM = 64
NSLOT = 2


def _pow2_reps(rows):
    r = 1
    while r * 2 * rows <= 2048 and r < 512:
        r *= 2
    return r


def kernel(x, comp_table, emb0, emb1, emb2, emb3, emb4):
    batch = x.shape[0]
    per_w = batch // NW
    n_group = per_w // L
    srcs = (emb0, emb1, emb2, emb3, emb4)
    n_rows = [e.shape[0] for e in srcs]
    reps = [_pow2_reps(b) for b in n_rows]
    embs = tuple(
        jnp.tile(e.at[0].set(0.0), (r, 1)) for e, r in zip(srcs, reps)
    )
    t_ids = jnp.arange(comp_table.shape[0], dtype=jnp.int32)[:, None]
    offs = (t_ids % jnp.asarray(reps, jnp.int32)) * jnp.asarray(n_rows, jnp.int32)
    comp_flat = (comp_table + offs).reshape(-1)

    def body(x_hbm, comp_hbm, e0, e1, e2, e3, e4,
             o0, o1, o2, o3, o4,
             x_v, cidx_v, cvals_v, rows_v, semc, semg, semw):
        ebs = (e0, e1, e2, e3, e4)
        outs = (o0, o1, o2, o3, o4)
        wid = lax.axis_index("s") * NC + lax.axis_index("c")
        base = wid * per_w

        pltpu.sync_copy(x_hbm.at[pl.ds(base, per_w)], x_v)
        for j in range(n_group):
            xg5 = x_v[pl.ds(j * L, L)] * N_COMP
            for i in range(N_COMP):
                cidx_v[pl.ds(i * per_w + j * L, L)] = xg5 + i

        comp_dmas = [
            pltpu.async_copy(
                comp_hbm.at[cidx_v.at[pl.ds(i * per_w, per_w)]],
                cvals_v.at[pl.ds(i * per_w, per_w)], semc)
            for i in range(N_COMP)
        ]
        for d in comp_dmas:
            d.wait()

        gd = [None] * N_COMP
        wd = [None] * N_COMP

        def fire(i):
            gd[i] = pltpu.async_copy(
                ebs[i].at[cvals_v.at[pl.ds(i * per_w, per_w)]],
                rows_v.at[i % NSLOT], semg.at[i % NSLOT])

        for i in range(min(NSLOT, N_COMP)):
            fire(i)
        for i in range(N_COMP):
            gd[i].wait()
            wd[i] = pltpu.async_copy(
                rows_v.at[i % NSLOT], outs[i].at[pl.ds(base, per_w)],
                semw.at[i % NSLOT])
            if i + NSLOT < N_COMP:
                wd[i].wait()
                fire(i + NSLOT)
        for i in range(max(0, N_COMP - NSLOT), N_COMP):
            wd[i].wait()

    mesh = plsc.VectorSubcoreMesh(core_axis_name="c", subcore_axis_name="s")
    out_type = tuple(
        jax.ShapeDtypeStruct((batch, C_DIM), jnp.float32) for _ in range(N_COMP)
    )
    scratch = [
        pltpu.VMEM((per_w,), jnp.int32),
        pltpu.VMEM((N_COMP * per_w,), jnp.int32),
        pltpu.VMEM((N_COMP * per_w,), jnp.int32),
        pltpu.VMEM((NSLOT, per_w, C_DIM), jnp.float32),
        pltpu.SemaphoreType.DMA,
        pltpu.SemaphoreType.DMA((NSLOT,)),
        pltpu.SemaphoreType.DMA((NSLOT,)),
    ]
    f = pl.kernel(
        body, mesh=mesh, out_type=out_type, scratch_types=scratch,
        compiler_params=pltpu.CompilerParams(use_tc_tiling_on_sc=False),
    )
    return f(x, comp_flat, *embs)


# speedup vs baseline: 1.0678x; 1.0678x over previous
import jax
import jax.numpy as jnp
from jax import lax
from jax.experimental import pallas as pl
from jax.experimental.pallas import tpu as pltpu
from jax.experimental.pallas import tpu_sc as plsc

L = 16
NC = 2
NS = 16
NW = NC * NS
N_COMP = 5
C_DIM = 64
NSLOT = 3


def _tce_body(x_hbm, comp_hbm, e0, e1, e2, e3, e4,
              o0, o1, o2, o3, o4,
              x_v, cidx_v, cvals_v, rows_v, semc, semg, semw):
    embs = (e0, e1, e2, e3, e4)
    outs = (o0, o1, o2, o3, o4)
    batch = x_hbm.shape[0]
    t_vocab = comp_hbm.shape[0] // N_COMP
    per_w = batch // NW

    wid = lax.axis_index("s") * NC + lax.axis_index("c")
    base = wid * per_w

    pltpu.sync_copy(x_hbm.at[pl.ds(base, per_w)], x_v)
    for i in range(N_COMP):
        off = jnp.int32(i * t_vocab)
        for j in range(per_w // L):
            cidx_v[pl.ds(i * per_w + j * L, L)] = x_v[pl.ds(j * L, L)] + off

    cg = [
        pltpu.async_copy(
            comp_hbm.at[cidx_v.at[pl.ds(i * per_w, per_w)]],
            cvals_v.at[pl.ds(i * per_w, per_w)], semc.at[i])
        for i in range(N_COMP)
    ]

    gd = [None] * N_COMP
    wd = [None] * N_COMP

    def fire(i):
        gd[i] = pltpu.async_copy(
            embs[i].at[cvals_v.at[pl.ds(i * per_w, per_w)]],
            rows_v.at[i % NSLOT], semg.at[i % NSLOT])

    def fire_wb(i):
        wd[i] = pltpu.async_copy(
            rows_v.at[i % NSLOT], outs[i].at[pl.ds(base, per_w)],
            semw.at[i % NSLOT])

    for i in range(N_COMP):
        if i >= NSLOT:
            wd[i - NSLOT].wait()
        cg[i].wait()
        fire(i)
        if i >= 1:
            gd[i - 1].wait()
            fire_wb(i - 1)
    gd[N_COMP - 1].wait()
    fire_wb(N_COMP - 1)
    for i in range(max(0, N_COMP - NSLOT), N_COMP):
        wd[i].wait()


def kernel(x, comp_table, emb0, emb1, emb2, emb3, emb4):
    batch = x.shape[0]
    per_w = batch // NW
    t_vocab = comp_table.shape[0]
    reps = [max(1, min(1024, 4096 // e.shape[0])) for e in
            (emb0, emb1, emb2, emb3, emb4)]
    embs = tuple(
        jnp.tile(e.at[0].set(0.0), (r, 1))
        for e, r in zip((emb0, emb1, emb2, emb3, emb4), reps)
    )
    t_ids = jnp.arange(t_vocab, dtype=jnp.int32)
    cols = [
        comp_table[:, i] + (t_ids % reps[i]) * e.shape[0]
        for i, e in enumerate((emb0, emb1, emb2, emb3, emb4))
    ]
    comp_cm = jnp.concatenate(cols)

    mesh = plsc.VectorSubcoreMesh(core_axis_name="c", subcore_axis_name="s")
    out_type = tuple(
        jax.ShapeDtypeStruct((batch, C_DIM), jnp.float32) for _ in range(N_COMP)
    )
    scratch = [
        pltpu.VMEM((per_w,), jnp.int32),
        pltpu.VMEM((N_COMP * per_w,), jnp.int32),
        pltpu.VMEM((N_COMP * per_w,), jnp.int32),
        pltpu.VMEM((NSLOT, per_w, C_DIM), jnp.float32),
        pltpu.SemaphoreType.DMA((N_COMP,)),
        pltpu.SemaphoreType.DMA((NSLOT,)),
        pltpu.SemaphoreType.DMA((NSLOT,)),
    ]
    f = pl.kernel(
        _tce_body, mesh=mesh, out_type=out_type, scratch_types=scratch,
        compiler_params=pltpu.CompilerParams(use_tc_tiling_on_sc=False),
    )
    return f(x, comp_cm, *embs)


# speedup vs baseline: 1.1358x; 1.0637x over previous
import jax
import jax.numpy as jnp
from jax import lax
from jax.experimental import pallas as pl
from jax.experimental.pallas import tpu as pltpu
from jax.experimental.pallas import tpu_sc as plsc

L = 16
NC = 2
NS = 16
NW = NC * NS
N_COMP = 5
C_DIM = 64
NSLOT = 3


def _make_body(n_comp, per_w):
    def body(x_hbm, comp_hbm, *rest):
        embs = rest[:n_comp]
        outs = rest[n_comp:2 * n_comp]
        x_v, cidx_v, cvals_v, rows_v, semc, semg, semw = rest[2 * n_comp:]
        t_vocab = comp_hbm.shape[0] // n_comp
        nslot = min(NSLOT, n_comp)

        wid = lax.axis_index("s") * NC + lax.axis_index("c")
        base = wid * per_w

        pltpu.sync_copy(x_hbm.at[pl.ds(base, per_w)], x_v)
        for i in range(n_comp):
            off = jnp.int32(i * t_vocab)
            for j in range(per_w // L):
                cidx_v[pl.ds(i * per_w + j * L, L)] = x_v[pl.ds(j * L, L)] + off

        cg = [
            pltpu.async_copy(
                comp_hbm.at[cidx_v.at[pl.ds(i * per_w, per_w)]],
                cvals_v.at[pl.ds(i * per_w, per_w)], semc.at[i])
            for i in range(n_comp)
        ]

        gd = [None] * n_comp
        wd = [None] * n_comp

        def fire(i):
            gd[i] = pltpu.async_copy(
                embs[i].at[cvals_v.at[pl.ds(i * per_w, per_w)]],
                rows_v.at[i % nslot], semg.at[i % nslot])

        def fire_wb(i):
            wd[i] = pltpu.async_copy(
                rows_v.at[i % nslot], outs[i].at[pl.ds(base, per_w)],
                semw.at[i % nslot])

        for i in range(n_comp):
            if i >= nslot:
                wd[i - nslot].wait()
            cg[i].wait()
            fire(i)
            if i >= 1:
                gd[i - 1].wait()
                fire_wb(i - 1)
        gd[n_comp - 1].wait()
        fire_wb(n_comp - 1)
        for i in range(max(0, n_comp - nslot), n_comp):
            wd[i].wait()

    return body


def _sc_call(x, comp_cm, embs, batch, per_w):
    n_comp = len(embs)
    mesh = plsc.VectorSubcoreMesh(core_axis_name="c", subcore_axis_name="s")
    out_type = tuple(
        jax.ShapeDtypeStruct((batch, C_DIM), jnp.float32) for _ in range(n_comp)
    )
    nslot = min(NSLOT, n_comp)
    scratch = [
        pltpu.VMEM((per_w,), jnp.int32),
        pltpu.VMEM((n_comp * per_w,), jnp.int32),
        pltpu.VMEM((n_comp * per_w,), jnp.int32),
        pltpu.VMEM((nslot, per_w, C_DIM), jnp.float32),
        pltpu.SemaphoreType.DMA((n_comp,)),
        pltpu.SemaphoreType.DMA((nslot,)),
        pltpu.SemaphoreType.DMA((nslot,)),
    ]
    f = pl.kernel(
        _make_body(n_comp, per_w), mesh=mesh, out_type=out_type,
        scratch_types=scratch,
        compiler_params=pltpu.CompilerParams(use_tc_tiling_on_sc=False),
    )
    return f(x, comp_cm, *embs)


def kernel(x, comp_table, emb0, emb1, emb2, emb3, emb4):
    batch = x.shape[0]
    per_w = batch // NW
    t_vocab = comp_table.shape[0]
    srcs = (emb0, emb1, emb2, emb3, emb4)
    reps = [max(1, min(1024, 4096 // e.shape[0])) for e in srcs]
    embs = tuple(
        jnp.tile(e.at[0].set(0.0), (r, 1)) for e, r in zip(srcs, reps)
    )
    t_ids = jnp.arange(t_vocab, dtype=jnp.int32)
    cols = [
        comp_table[:, i] + (t_ids % reps[i]) * e.shape[0]
        for i, e in enumerate(srcs)
    ]
    cm_a = jnp.concatenate(cols[:2])
    cm_b = jnp.concatenate(cols[2:])
    out_a = _sc_call(x, cm_a, embs[:2], batch, per_w)
    out_b = _sc_call(x, cm_b, embs[2:], batch, per_w)
    return out_a + out_b
